# Initial kernel scaffold; baseline (speedup 1.0000x reference)
import jax
import jax.numpy as jnp
from jax.experimental import pallas as pl


def kernel(scores, box_deltas, anchors, img_scale):
    raise NotImplementedError("write your pallas kernel here")


# speedup vs baseline: 11.6530x; 11.6530x over previous
import jax
import jax.numpy as jnp
from jax.experimental import pallas as pl

_N = 5000
_PRE_NMS = 1000
_MAX_DET = 100
_IOU_THRESH = 0.5
_ROWS = 40
_LANES = 128
_PAD = _ROWS * _LANES
_NEG = -1e30


def _nms_kernel(s_ref, ty_ref, tx_ref, th_ref, tw_ref,
                a0_ref, a1_ref, a2_ref, a3_ref, scale_ref, out_ref):
    s2d = s_ref[...]
    scale = scale_ref[0, 0]

    row_i = jax.lax.broadcasted_iota(jnp.int32, (_ROWS, _LANES), 0)
    lane_i = jax.lax.broadcasted_iota(jnp.int32, (_ROWS, _LANES), 1)
    flat_idx = row_i * _LANES + lane_i

    s3 = s2d[:, :, None]
    flat3 = flat_idx[:, :, None]
    lane_row = jax.lax.broadcasted_iota(jnp.int32, (1, 1, _LANES), 2)

    def rank_body(jb, counts):
        pj = s_ref[pl.ds(jb, 1), :]
        pj3 = pj[:, None, :]
        jidx3 = jb * _LANES + lane_row
        beats = (pj3 > s3) | ((pj3 == s3) & (jidx3 < flat3))
        return counts + jnp.sum(beats.astype(jnp.int32), axis=2)

    counts = jax.lax.fori_loop(
        0, _ROWS, rank_body, jnp.zeros((_ROWS, _LANES), jnp.int32))
    candidate = counts < _PRE_NMS

    a0 = a0_ref[...]
    a1 = a1_ref[...]
    a2 = a2_ref[...]
    a3 = a3_ref[...]
    ya = (a0 + a2) * 0.5
    xa = (a1 + a3) * 0.5
    ha = a2 - a0
    wa = a3 - a1
    yc = ty_ref[...] * ha + ya
    xc = tx_ref[...] * wa + xa
    h = jnp.exp(th_ref[...]) * ha
    w = jnp.exp(tw_ref[...]) * wa
    by1 = yc - h * 0.5
    bx1 = xc - w * 0.5
    by2 = yc + h * 0.5
    bx2 = xc + w * 0.5
    area = (by2 - by1) * (bx2 - bx1)

    probs = jax.nn.sigmoid(s2d)
    scored0 = jnp.where(candidate, probs, _NEG)

    out_row = jax.lax.broadcasted_iota(jnp.int32, (_LANES, 8), 0)
    out_col = jax.lax.broadcasted_iota(jnp.int32, (_LANES, 8), 1)

    def nms_body(i, carry):
        scored, out = carry
        m = jnp.max(scored)
        sel_any = scored == m
        idx = jnp.min(jnp.where(sel_any, flat_idx, 2 ** 30))
        sel = flat_idx == idx

        def pick(arr):
            return jnp.sum(jnp.where(sel, arr, 0.0))

        y1s = pick(by1)
        x1s = pick(bx1)
        y2s = pick(by2)
        x2s = pick(bx2)
        ar_s = pick(area)

        is_real = m > 0.0
        yy1 = jnp.maximum(y1s, by1)
        xx1 = jnp.maximum(x1s, bx1)
        yy2 = jnp.minimum(y2s, by2)
        xx2 = jnp.minimum(x2s, bx2)
        inter = jnp.clip(yy2 - yy1, 0.0) * jnp.clip(xx2 - xx1, 0.0)
        iou = inter / (ar_s + area - inter + 1e-8)

        suppress = is_real & (iou > _IOU_THRESH) & (scored > 0.0)
        scored = jnp.where(suppress, scored - 2.0, scored)
        scored = jnp.where(sel, _NEG, scored)

        out_s = jnp.where(is_real, m, -1.0)
        rowvals = jnp.where(out_col == 0, y1s * scale,
                  jnp.where(out_col == 1, x1s * scale,
                  jnp.where(out_col == 2, y2s * scale,
                  jnp.where(out_col == 3, x2s * scale,
                  jnp.where(out_col == 4, out_s, 0.0)))))
        out = jnp.where(out_row == i, rowvals, out)
        return scored, out

    _, out = jax.lax.fori_loop(
        0, _MAX_DET, nms_body,
        (scored0, jnp.zeros((_LANES, 8), jnp.float32)))
    out_ref[...] = out


def kernel(scores, box_deltas, anchors, img_scale):
    pad = _PAD - _N
    s = jnp.pad(scores, (0, pad), constant_values=_NEG).reshape(_ROWS, _LANES)
    bd = jnp.pad(box_deltas, ((0, pad), (0, 0)))
    an = jnp.pad(anchors, ((0, pad), (0, 0)))
    planes = [bd[:, k].reshape(_ROWS, _LANES) for k in range(4)]
    planes += [an[:, k].reshape(_ROWS, _LANES) for k in range(4)]
    scale = img_scale.reshape(1, 1)
    out = pl.pallas_call(
        _nms_kernel,
        out_shape=jax.ShapeDtypeStruct((_LANES, 8), jnp.float32),
    )(s, *planes, scale)
    return out[:_MAX_DET, :5]


# speedup vs baseline: 12.8916x; 1.1063x over previous
import jax
import jax.numpy as jnp
from jax.experimental import pallas as pl

_N = 5000
_PRE_NMS = 1000
_MAX_DET = 100
_IOU_THRESH = 0.5
_ROWS = 40
_LANES = 128
_PAD = _ROWS * _LANES
_NEG = -1e30


def _nms_kernel(s_ref, ty_ref, tx_ref, th_ref, tw_ref,
                a0_ref, a1_ref, a2_ref, a3_ref, scale_ref, out_ref):
    s2d = s_ref[...]
    scale = scale_ref[0, 0]

    row_i = jax.lax.broadcasted_iota(jnp.int32, (_ROWS, _LANES), 0)
    lane_i = jax.lax.broadcasted_iota(jnp.int32, (_ROWS, _LANES), 1)
    flat_idx = row_i * _LANES + lane_i

    s3 = s2d[:, :, None]
    flat3 = flat_idx[:, :, None]
    lane_row = jax.lax.broadcasted_iota(jnp.int32, (1, 1, _LANES), 2)

    def rank_body(jb, counts):
        pj = s_ref[pl.ds(jb, 1), :]
        pj3 = pj[:, None, :]
        jidx3 = jb * _LANES + lane_row
        beats = (pj3 > s3) | ((pj3 == s3) & (jidx3 < flat3))
        return counts + jnp.sum(beats.astype(jnp.int32), axis=2)

    counts = jax.lax.fori_loop(
        0, _ROWS, rank_body, jnp.zeros((_ROWS, _LANES), jnp.int32),
        unroll=4)
    candidate = counts < _PRE_NMS

    a0 = a0_ref[...]
    a1 = a1_ref[...]
    a2 = a2_ref[...]
    a3 = a3_ref[...]
    ya = (a0 + a2) * 0.5
    xa = (a1 + a3) * 0.5
    ha = a2 - a0
    wa = a3 - a1
    yc = ty_ref[...] * ha + ya
    xc = tx_ref[...] * wa + xa
    h = jnp.exp(th_ref[...]) * ha
    w = jnp.exp(tw_ref[...]) * wa
    by1 = yc - h * 0.5
    bx1 = xc - w * 0.5
    by2 = yc + h * 0.5
    bx2 = xc + w * 0.5
    area = (by2 - by1) * (bx2 - bx1)

    probs = jax.nn.sigmoid(s2d)
    scored0 = jnp.where(candidate, probs, _NEG)

    out_row = jax.lax.broadcasted_iota(jnp.int32, (_LANES, 8), 0)
    out_col = jax.lax.broadcasted_iota(jnp.int32, (_LANES, 8), 1)

    def nms_body(i, carry):
        scored, out = carry
        m = jnp.max(scored)
        sel_any = scored == m
        idx = jnp.min(jnp.where(sel_any, flat_idx, 2 ** 30))
        sel = flat_idx == idx

        def pick(arr):
            return jnp.sum(jnp.where(sel, arr, 0.0))

        y1s = pick(by1)
        x1s = pick(bx1)
        y2s = pick(by2)
        x2s = pick(bx2)
        ar_s = pick(area)

        is_real = m > 0.0
        yy1 = jnp.maximum(y1s, by1)
        xx1 = jnp.maximum(x1s, bx1)
        yy2 = jnp.minimum(y2s, by2)
        xx2 = jnp.minimum(x2s, bx2)
        inter = jnp.clip(yy2 - yy1, 0.0) * jnp.clip(xx2 - xx1, 0.0)
        iou = inter / (ar_s + area - inter + 1e-8)

        suppress = is_real & (iou > _IOU_THRESH) & (scored > 0.0)
        scored = jnp.where(suppress, scored - 2.0, scored)
        scored = jnp.where(sel, _NEG, scored)

        out_s = jnp.where(is_real, m, -1.0)
        rowvals = jnp.where(out_col == 0, y1s * scale,
                  jnp.where(out_col == 1, x1s * scale,
                  jnp.where(out_col == 2, y2s * scale,
                  jnp.where(out_col == 3, x2s * scale,
                  jnp.where(out_col == 4, out_s, 0.0)))))
        out = jnp.where(out_row == i, rowvals, out)
        return scored, out

    _, out = jax.lax.fori_loop(
        0, _MAX_DET, nms_body,
        (scored0, jnp.zeros((_LANES, 8), jnp.float32)),
        unroll=2)
    out_ref[...] = out


def kernel(scores, box_deltas, anchors, img_scale):
    pad = _PAD - _N
    s = jnp.pad(scores, (0, pad), constant_values=_NEG).reshape(_ROWS, _LANES)
    bd = jnp.pad(box_deltas, ((0, pad), (0, 0)))
    an = jnp.pad(anchors, ((0, pad), (0, 0)))
    planes = [bd[:, k].reshape(_ROWS, _LANES) for k in range(4)]
    planes += [an[:, k].reshape(_ROWS, _LANES) for k in range(4)]
    scale = img_scale.reshape(1, 1)
    out = pl.pallas_call(
        _nms_kernel,
        out_shape=jax.ShapeDtypeStruct((_LANES, 8), jnp.float32),
    )(s, *planes, scale)
    return out[:_MAX_DET, :5]


# speedup vs baseline: 13.0999x; 1.0162x over previous
import jax
import jax.numpy as jnp
from jax.experimental import pallas as pl

_N = 5000
_PRE_NMS = 1000
_MAX_DET = 100
_IOU_THRESH = 0.5
_ROWS = 40
_LANES = 128
_PAD = _ROWS * _LANES
_NEG = -1e30


def _nms_kernel(s_ref, ty_ref, tx_ref, th_ref, tw_ref,
                a0_ref, a1_ref, a2_ref, a3_ref, scale_ref, out_ref):
    s2d = s_ref[...]
    scale = scale_ref[0, 0]

    row_i = jax.lax.broadcasted_iota(jnp.int32, (_ROWS, _LANES), 0)
    lane_i = jax.lax.broadcasted_iota(jnp.int32, (_ROWS, _LANES), 1)
    flat_idx = row_i * _LANES + lane_i

    s3 = s2d[:, :, None]
    flat3 = flat_idx[:, :, None]
    lane_row = jax.lax.broadcasted_iota(jnp.int32, (1, 1, _LANES), 2)

    def rank_body(jb, counts):
        pj = s_ref[pl.ds(jb, 1), :]
        pj3 = pj[:, None, :]
        jidx3 = jb * _LANES + lane_row
        beats = (pj3 > s3) | ((pj3 == s3) & (jidx3 < flat3))
        return counts + jnp.sum(beats.astype(jnp.int32), axis=2)

    counts = jax.lax.fori_loop(
        0, _ROWS, rank_body, jnp.zeros((_ROWS, _LANES), jnp.int32),
        unroll=8)
    candidate = counts < _PRE_NMS

    a0 = a0_ref[...]
    a1 = a1_ref[...]
    a2 = a2_ref[...]
    a3 = a3_ref[...]
    ya = (a0 + a2) * 0.5
    xa = (a1 + a3) * 0.5
    ha = a2 - a0
    wa = a3 - a1
    yc = ty_ref[...] * ha + ya
    xc = tx_ref[...] * wa + xa
    h = jnp.exp(th_ref[...]) * ha
    w = jnp.exp(tw_ref[...]) * wa
    by1 = yc - h * 0.5
    bx1 = xc - w * 0.5
    by2 = yc + h * 0.5
    bx2 = xc + w * 0.5
    area = (by2 - by1) * (bx2 - bx1)

    probs = jax.nn.sigmoid(s2d)
    scored0 = jnp.where(candidate, probs, _NEG)

    out_row = jax.lax.broadcasted_iota(jnp.int32, (_LANES, 8), 0)
    out_col = jax.lax.broadcasted_iota(jnp.int32, (_LANES, 8), 1)

    def nms_body(i, carry):
        scored, out = carry
        m = jnp.max(scored)
        sel_any = scored == m
        idx = jnp.min(jnp.where(sel_any, flat_idx, 2 ** 30))
        sel = flat_idx == idx

        def pick(arr):
            return jnp.sum(jnp.where(sel, arr, 0.0))

        y1s = pick(by1)
        x1s = pick(bx1)
        y2s = pick(by2)
        x2s = pick(bx2)
        ar_s = pick(area)

        is_real = m > 0.0
        yy1 = jnp.maximum(y1s, by1)
        xx1 = jnp.maximum(x1s, bx1)
        yy2 = jnp.minimum(y2s, by2)
        xx2 = jnp.minimum(x2s, bx2)
        inter = jnp.clip(yy2 - yy1, 0.0) * jnp.clip(xx2 - xx1, 0.0)
        iou = inter / (ar_s + area - inter + 1e-8)

        suppress = is_real & (iou > _IOU_THRESH) & (scored > 0.0)
        scored = jnp.where(suppress, scored - 2.0, scored)
        scored = jnp.where(sel, _NEG, scored)

        out_s = jnp.where(is_real, m, -1.0)
        rowvals = jnp.where(out_col == 0, y1s * scale,
                  jnp.where(out_col == 1, x1s * scale,
                  jnp.where(out_col == 2, y2s * scale,
                  jnp.where(out_col == 3, x2s * scale,
                  jnp.where(out_col == 4, out_s, 0.0)))))
        out = jnp.where(out_row == i, rowvals, out)
        return scored, out

    _, out = jax.lax.fori_loop(
        0, _MAX_DET, nms_body,
        (scored0, jnp.zeros((_LANES, 8), jnp.float32)),
        unroll=4)
    out_ref[...] = out


def kernel(scores, box_deltas, anchors, img_scale):
    pad = _PAD - _N
    s = jnp.pad(scores, (0, pad), constant_values=_NEG).reshape(_ROWS, _LANES)
    bd = jnp.pad(box_deltas, ((0, pad), (0, 0)))
    an = jnp.pad(anchors, ((0, pad), (0, 0)))
    planes = [bd[:, k].reshape(_ROWS, _LANES) for k in range(4)]
    planes += [an[:, k].reshape(_ROWS, _LANES) for k in range(4)]
    scale = img_scale.reshape(1, 1)
    out = pl.pallas_call(
        _nms_kernel,
        out_shape=jax.ShapeDtypeStruct((_LANES, 8), jnp.float32),
    )(s, *planes, scale)
    return out[:_MAX_DET, :5]
